# baseline (speedup 1.0000x reference)
import functools

import jax
import jax.numpy as jnp
from jax import lax
from jax.experimental import pallas as pl
from jax.experimental.pallas import tpu as pltpu
from jax.experimental.pallas import tpu_sc as plsc

_NC = 2
_NS = 16
_NW = _NC * _NS
_CH = 128
_MW = 128


def _mesh():
    return plsc.VectorSubcoreMesh(
        core_axis_name="c", subcore_axis_name="s",
        num_cores=_NC, num_subcores=_NS)


def _prep_body(x_ref, wd_ref, ws_ref, td_ref, ts_ref):
    xv = x_ref[...]
    td_ref[...] = jnp.dot(xv, wd_ref[...], preferred_element_type=jnp.float32)
    ts_ref[...] = jnp.dot(xv, ws_ref[...], preferred_element_type=jnp.float32)


def _prep(x, wd, ws):
    n = x.shape[0]
    h2 = wd.shape[1]
    return pl.pallas_call(
        _prep_body,
        out_shape=(jax.ShapeDtypeStruct((n, h2), jnp.float32),
                   jax.ShapeDtypeStruct((n, h2), jnp.float32)),
    )(x, wd, ws)


def _sc_gather(td, ts, pvflat, src, dst):
    n, h2 = td.shape
    e = src.shape[0]
    ncht = e // _CH

    nte = (ncht // (2 * _NW)) * (2 * _NW)
    ntw = nte // _NW
    nextra = ncht - nte

    @functools.partial(
        pl.kernel, mesh=_mesh(),
        compiler_params=pltpu.CompilerParams(needs_layout_passes=False),
        out_type=(jax.ShapeDtypeStruct((e, h2), jnp.float32),
                  jax.ShapeDtypeStruct((8, e), jnp.float32)),
        scratch_types=[pltpu.VMEM((2, _CH), jnp.int32),
                       pltpu.VMEM((2, _CH), jnp.int32),
                       pltpu.VMEM((2, _CH, h2), jnp.float32),
                       pltpu.VMEM((2, _CH, h2), jnp.float32),
                       pltpu.VMEM((2, 8, _CH), jnp.float32),
                       pltpu.VMEM((4 * n,), jnp.float32),
                       pltpu.SemaphoreType.DMA,
                       pltpu.SemaphoreType.DMA,
                       pltpu.SemaphoreType.DMA],
    )
    def k(td_h, ts_h, pv_h, src_h, dst_h, s_o, geo_o,
          idxs, idxd, bufd, bufs, gbuf, pvt, semg, semo, semi):
        wid = lax.axis_index("s") * _NC + lax.axis_index("c")
        pltpu.sync_copy(pv_h, pvt)

        def fire_idx(t, b):
            base = (wid + t * _NW) * _CH
            pltpu.async_copy(src_h.at[pl.ds(base, _CH)], idxs.at[b], semi)
            pltpu.async_copy(dst_h.at[pl.ds(base, _CH)], idxd.at[b], semi)

        def wait_idx(t, b):
            base = (wid + t * _NW) * _CH
            pltpu.make_async_copy(
                src_h.at[pl.ds(base, _CH)], idxs.at[b], semi).wait()
            pltpu.make_async_copy(
                dst_h.at[pl.ds(base, _CH)], idxd.at[b], semi).wait()

        def fire_rows(b):
            pltpu.async_copy(td_h.at[idxd.at[b]], bufd.at[b], semg)
            pltpu.async_copy(ts_h.at[idxs.at[b]], bufs.at[b], semg)

        def fire(t, b):
            fire_idx(t, b)
            wait_idx(t, b)
            fire_rows(b)

        def compute(b):
            @plsc.parallel_loop(0, _CH // 16)
            def _(g):
                sl = pl.ds(g * 16, 16)
                s4 = idxs[b, sl] * 4
                d4 = idxd[b, sl] * 4
                rpx = plsc.load_gather(pvt, [s4]) - plsc.load_gather(pvt, [d4])
                rpy = (plsc.load_gather(pvt, [s4 + 1])
                       - plsc.load_gather(pvt, [d4 + 1]))
                rvx = (plsc.load_gather(pvt, [s4 + 2])
                       - plsc.load_gather(pvt, [d4 + 2]))
                rvy = (plsc.load_gather(pvt, [s4 + 3])
                       - plsc.load_gather(pvt, [d4 + 3]))
                dsq = rpx * rpx + rpy * rpy
                dvr = rvx * rpx + rvy * rpy
                r2 = jnp.minimum(1.0 / (dsq + 0.05), 20.0)
                r6 = jnp.minimum(r2 * r2 * r2, 400.0)
                r12 = jnp.minimum(r6 * r6, 160000.0)
                gbuf[b, 0, sl] = dsq
                gbuf[b, 1, sl] = dvr
                gbuf[b, 2, sl] = r2
                gbuf[b, 3, sl] = r6
                gbuf[b, 4, sl] = r12
                gbuf[b, 5, sl] = rpx
                gbuf[b, 6, sl] = rpy
                gbuf[b, 7, sl] = rvx

            @plsc.parallel_loop(0, _CH, unroll=4)
            def _(r):
                for cc in range(h2 // 16):
                    sl = pl.ds(cc * 16, 16)
                    bufd[b, r, sl] = bufd[b, r, sl] + bufs[b, r, sl]

        def drain_out(b, base):
            pltpu.make_async_copy(
                bufd.at[b], s_o.at[pl.ds(base, _CH)], semo).wait()
            pltpu.make_async_copy(
                gbuf.at[b], geo_o.at[:, pl.ds(base, _CH)], semo).wait()

        fire(0, 0)
        fire_idx(1, 1)

        def pair(p, carry):
            for b in range(2):
                t = 2 * p + b
                base = (wid + t * _NW) * _CH

                @pl.when(t >= 2)
                def _():
                    drain_out(b, base)

                @pl.when(t + 1 < ntw)
                def _():
                    wait_idx(t + 1, 1 - b)
                    fire_rows(1 - b)

                pltpu.make_async_copy(
                    td_h.at[idxd.at[b]], bufd.at[b], semg).wait()
                pltpu.make_async_copy(
                    ts_h.at[idxs.at[b]], bufs.at[b], semg).wait()
                compute(b)

                @pl.when(t + 2 < ntw)
                def _():
                    fire_idx(t + 2, b)

                pltpu.async_copy(bufd.at[b], s_o.at[pl.ds(base, _CH)], semo)
                pltpu.async_copy(gbuf.at[b], geo_o.at[:, pl.ds(base, _CH)],
                                 semo)
            return carry

        lax.fori_loop(0, ntw // 2, pair, 0)
        for b in range(2):
            drain_out(b, wid * _CH)

        @pl.when(wid < nextra)
        def _():
            base = (wid + ntw * _NW) * _CH
            fire(ntw, 0)
            pltpu.make_async_copy(
                td_h.at[idxd.at[0]], bufd.at[0], semg).wait()
            pltpu.make_async_copy(
                ts_h.at[idxs.at[0]], bufs.at[0], semg).wait()
            compute(0)
            pltpu.sync_copy(bufd.at[0], s_o.at[pl.ds(base, _CH)])
            pltpu.sync_copy(gbuf.at[0], geo_o.at[:, pl.ds(base, _CH)])

    return k(td, ts, pvflat, src, dst)


def _sp(x):
    return jnp.maximum(x, 0.0) + jnp.log1p(jnp.exp(-jnp.abs(x)))


def _ln(t, g, b):
    mu = jnp.mean(t, axis=1, keepdims=True)
    var = jnp.mean((t - mu) ** 2, axis=1, keepdims=True)
    return (t - mu) * lax.rsqrt(var + 1e-5) * g + b


def _edge_body(s_ref, geo_ref, wg_ref, bcat_ref, mm_ref, lncg_ref, lncb_ref,
               w1c_ref, b1c_ref, w2c_ref, b2c_ref, m_ref):
    s = s_ref[...]
    hh = s.shape[1] // 2
    be = s.shape[0]
    geo = jnp.transpose(geo_ref[...])
    rel_pos = geo[:, 5:7]
    u = s + bcat_ref[...] + jnp.dot(geo, wg_ref[...],
                                    preferred_element_type=jnp.float32)
    mm = mm_ref[...]
    mu = jnp.dot(u, mm, preferred_element_type=jnp.float32)
    du = u - mu
    var = jnp.dot(du * du, mm, preferred_element_type=jnp.float32)
    a1 = _sp(du * lax.rsqrt(var + 1e-5) * lncg_ref[...] + lncb_ref[...])
    t = (jnp.dot(a1, w1c_ref[...], preferred_element_type=jnp.float32)
         + b1c_ref[...])
    w_edge = t[:, hh:hh + 1]
    m_h = (jnp.dot(_sp(t), w2c_ref[...], preferred_element_type=jnp.float32)
           + b2c_ref[...])
    m_v = w_edge * rel_pos
    m_ref[...] = m_h + jnp.concatenate(
        [jnp.zeros((be, hh), jnp.float32), m_v,
         jnp.ones((be, 1), jnp.float32),
         jnp.zeros((be, _MW - hh - 3), jnp.float32)], axis=1)


def _edge(s, geo, wg, bcat, mm, lncg, lncb, w1c, b1c, w2c, b2c):
    e, h2 = s.shape
    be = 2560
    grid = (e // be,)
    full = lambda a: pl.BlockSpec(a.shape, lambda i: (0,) * a.ndim)
    ws = [wg, bcat, mm, lncg, lncb, w1c, b1c, w2c, b2c]
    return pl.pallas_call(
        _edge_body,
        grid=grid,
        in_specs=[pl.BlockSpec((be, h2), lambda i: (i, 0)),
                  pl.BlockSpec((8, be), lambda i: (0, i))] +
                 [full(a) for a in ws],
        out_specs=pl.BlockSpec((be, _MW), lambda i: (i, 0)),
        out_shape=jax.ShapeDtypeStruct((e, _MW), jnp.float32),
    )(s, geo, *ws)


def _sc_scatter(ms, dst, n):
    e = dst.shape[0]
    ncht = e // _CH
    rows_t = (n // (8 * _NS)) * 8
    extra = n - _NS * rows_t
    zch = 104
    nz = rows_t // zch
    assert nz * zch == rows_t and extra <= zch

    qch = [m.shape[0] // _CH for m in ms]
    qoff = [sum(qch[:i]) for i in range(len(ms))]

    @functools.partial(
        pl.kernel, mesh=_mesh(),
        out_type=jax.ShapeDtypeStruct((_NC, n, _MW), jnp.float32),
        scratch_types=[pltpu.VMEM((2, _CH), jnp.int32),
                       pltpu.VMEM((2, _CH, _MW), jnp.float32),
                       pltpu.VMEM((zch, _MW), jnp.float32),
                       pltpu.VMEM_SHARED((n, _MW), jnp.float32),
                       pltpu.SemaphoreType.DMA],
    )
    def k(*refs):
        m_hs = refs[:len(ms)]
        dst_h, p_o, idxd, mbuf, zbuf, acc, semf = refs[len(ms):]
        cid = lax.axis_index("c")
        sid = lax.axis_index("s")
        wid = sid * _NC + cid

        def zrow(r, carry):
            for cc in range(_MW // 16):
                zbuf[r, pl.ds(cc * 16, 16)] = jnp.zeros((16,), jnp.float32)
            return carry

        lax.fori_loop(0, zch, zrow, 0)
        for j in range(nz):
            pltpu.sync_copy(zbuf, acc.at[pl.ds(sid * rows_t + j * zch, zch)])

        @pl.when(sid == 0)
        def _():
            pltpu.sync_copy(zbuf.at[pl.ds(0, extra)],
                            acc.at[pl.ds(_NS * rows_t, extra)])

        plsc.subcore_barrier()

        for q, m_h in enumerate(m_hs):
            ncq = qch[q]
            nte = (ncq // (2 * _NW)) * (2 * _NW)
            ntw = nte // _NW
            nextra = ncq - nte
            goff = qoff[q] * _CH

            def fire(t, b):
                base = (wid + t * _NW) * _CH
                pltpu.async_copy(dst_h.at[pl.ds(goff + base, _CH)],
                                 idxd.at[b], semf)
                pltpu.async_copy(m_h.at[pl.ds(base, _CH)], mbuf.at[b], semf)

            def wait_fire(t, b):
                base = (wid + t * _NW) * _CH
                pltpu.make_async_copy(
                    dst_h.at[pl.ds(goff + base, _CH)], idxd.at[b],
                    semf).wait()
                pltpu.make_async_copy(
                    m_h.at[pl.ds(base, _CH)], mbuf.at[b], semf).wait()

            fire(0, 0)

            def pair(p, carry):
                for b in range(2):
                    t = 2 * p + b

                    @pl.when(t + 1 < ntw)
                    def _():
                        fire(t + 1, 1 - b)

                    wait_fire(t, b)
                    pltpu.sync_copy(mbuf.at[b], acc.at[idxd.at[b]], add=True)
                return carry

            lax.fori_loop(0, ntw // 2, pair, 0)

            @pl.when(wid < nextra)
            def _():
                fire(ntw, 0)
                wait_fire(ntw, 0)
                pltpu.sync_copy(mbuf.at[0], acc.at[idxd.at[0]], add=True)

        plsc.subcore_barrier()
        pltpu.sync_copy(acc.at[pl.ds(sid * rows_t, rows_t)],
                        p_o.at[cid, pl.ds(sid * rows_t, rows_t)])

        @pl.when(sid == 0)
        def _():
            pltpu.sync_copy(acc.at[pl.ds(_NS * rows_t, extra)],
                            p_o.at[cid, pl.ds(_NS * rows_t, extra)])

    return k(*ms, dst)


def _dec_body(x_ref, p_ref, pb_ref, w0x_ref, w0a_ref, w0n_ref, hb0_ref,
              hlng_ref, hlnb_ref, hw1_ref, hb1_ref, o_ref):
    xv = x_ref[...]
    hh = w0a_ref.shape[0]
    p = (p_ref[0] + p_ref[1]) + (pb_ref[0] + pb_ref[1])
    cnt = jnp.maximum(p[:, hh + 2:hh + 3], 1.0)
    agg_h = p[:, 0:hh] / cnt
    agg_v = p[:, hh:hh + 2] / cnt + 1e-8
    mvn = jnp.sqrt(jnp.sum(agg_v * agg_v, axis=1, keepdims=True))
    t = (jnp.dot(xv, w0x_ref[...], preferred_element_type=jnp.float32)
         + jnp.dot(agg_h, w0a_ref[...], preferred_element_type=jnp.float32)
         + mvn * w0n_ref[...] + hb0_ref[...])
    t = jax.nn.softplus(_ln(t, hlng_ref[...], hlnb_ref[...]))
    o_ref[...] = (xv
                  + jnp.dot(t, hw1_ref[...], preferred_element_type=jnp.float32)
                  + hb1_ref[...])


def _decode(x, pa, pb, w0x, w0a, w0n, hb0, hlng, hlnb, hw1, hb1):
    n, d = x.shape
    bn = 2000
    grid = (n // bn,)
    full = lambda a: pl.BlockSpec(a.shape, lambda i: (0,) * a.ndim)
    ws = [w0x, w0a, w0n, hb0, hlng, hlnb, hw1, hb1]
    return pl.pallas_call(
        _dec_body,
        grid=grid,
        in_specs=[pl.BlockSpec((bn, d), lambda i: (i, 0)),
                  pl.BlockSpec((_NC, bn, _MW), lambda i: (0, i, 0)),
                  pl.BlockSpec((_NC, bn, _MW), lambda i: (0, i, 0))] +
                 [full(a) for a in ws],
        out_specs=pl.BlockSpec((bn, d), lambda i: (i, 0)),
        out_shape=jax.ShapeDtypeStruct((n, d), jnp.float32),
    )(x, pa, pb, *ws)


def kernel(x, pos, vel, edge_index,
           e_w0, e_b0, e_ln_g, e_ln_b, e_w1, e_b1, e_w2, e_b2,
           v_w0, v_b0, v_ln_g, v_ln_b, v_w1, v_b1,
           h_w0, h_b0, h_ln_g, h_ln_b, h_w1, h_b1):
    n, d = x.shape
    hh = e_b0.shape[0]
    src = edge_index[0].astype(jnp.int32)
    dst = edge_index[1].astype(jnp.int32)

    wd = jnp.concatenate([e_w0[:d], v_w0[:d]], axis=1)
    ws_ = jnp.concatenate([e_w0[d:2 * d], v_w0[d:2 * d]], axis=1)
    wg = jnp.concatenate([e_w0[2 * d:], v_w0[2 * d:]], axis=1)
    wg = jnp.concatenate([wg, jnp.zeros((3, 2 * hh), jnp.float32)], axis=0)
    bcat = jnp.concatenate([e_b0, v_b0]).reshape(1, 2 * hh)
    pvflat = jnp.concatenate([pos, vel], axis=1).reshape(-1)

    h2 = 2 * hh
    zb = jnp.zeros((hh, hh), jnp.float32)
    jb = jnp.full((hh, hh), 1.0 / hh, jnp.float32)
    mm = jnp.concatenate([jnp.concatenate([jb, zb], 1),
                          jnp.concatenate([zb, jb], 1)], 0)
    lncg = jnp.concatenate([e_ln_g, v_ln_g]).reshape(1, h2)
    lncb = jnp.concatenate([e_ln_b, v_ln_b]).reshape(1, h2)
    vw1p = jnp.concatenate([v_w1, jnp.zeros((hh, hh - 1), jnp.float32)], 1)
    w1c = jnp.concatenate([jnp.concatenate([e_w1, zb], 1),
                           jnp.concatenate([zb, vw1p], 1)], 0)
    b1c = jnp.concatenate(
        [e_b1, v_b1, jnp.zeros((hh - 1,), jnp.float32)]).reshape(1, h2)
    w2c = jnp.concatenate([jnp.concatenate([e_w2, zb], 1),
                           jnp.concatenate([zb, zb], 1)], 0)
    b2c = jnp.concatenate([e_b2, jnp.zeros((hh,), jnp.float32)]).reshape(1, h2)

    td, ts = _prep(x, wd, ws_)
    e = src.shape[0]
    ncht = e // _CH
    sl_ch = [320, 640, 640, 640, ncht - 2240]
    nspl = 3
    bounds = [0]
    for c in sl_ch:
        bounds.append(bounds[-1] + c * _CH)
    ms = []
    for q in range(len(sl_ch)):
        lo, hi = bounds[q], bounds[q + 1]
        s_q, geo_q = _sc_gather(td, ts, pvflat, src[lo:hi], dst[lo:hi])
        ms.append(_edge(s_q, geo_q, wg, bcat, mm, lncg, lncb,
                        w1c, b1c, w2c, b2c))
    mid = bounds[nspl]
    pa = _sc_scatter(ms[:nspl], dst[:mid], n)
    pb = _sc_scatter(ms[nspl:], dst[mid:], n)
    out = _decode(x, pa, pb, h_w0[:d], h_w0[d:d + hh],
                  h_w0[d + hh:].reshape(1, hh),
                  h_b0.reshape(1, hh), h_ln_g.reshape(1, hh),
                  h_ln_b.reshape(1, hh), h_w1, h_b1.reshape(1, d))
    return out


# speedup vs baseline: 1.0367x; 1.0367x over previous
import functools

import jax
import jax.numpy as jnp
from jax import lax
from jax.experimental import pallas as pl
from jax.experimental.pallas import tpu as pltpu
from jax.experimental.pallas import tpu_sc as plsc

_NC = 2
_NS = 16
_NW = _NC * _NS
_CH = 128
_MW = 128


def _mesh():
    return plsc.VectorSubcoreMesh(
        core_axis_name="c", subcore_axis_name="s",
        num_cores=_NC, num_subcores=_NS)


def _prep_body(x_ref, wd_ref, ws_ref, td_ref, ts_ref):
    xv = x_ref[...]
    td_ref[...] = jnp.dot(xv, wd_ref[...], preferred_element_type=jnp.float32)
    ts_ref[...] = jnp.dot(xv, ws_ref[...], preferred_element_type=jnp.float32)


def _prep(x, wd, ws):
    n = x.shape[0]
    h2 = wd.shape[1]
    return pl.pallas_call(
        _prep_body,
        out_shape=(jax.ShapeDtypeStruct((n, h2), jnp.float32),
                   jax.ShapeDtypeStruct((n, h2), jnp.float32)),
    )(x, wd, ws)


def _sc_gather(td, ts, pvflat, src, dst):
    n, h2 = td.shape
    e = src.shape[0]
    ncht = e // _CH

    nte = (ncht // (2 * _NW)) * (2 * _NW)
    ntw = nte // _NW
    nextra = ncht - nte

    @functools.partial(
        pl.kernel, mesh=_mesh(),
        compiler_params=pltpu.CompilerParams(needs_layout_passes=False),
        out_type=(jax.ShapeDtypeStruct((e, h2), jnp.float32),
                  jax.ShapeDtypeStruct((8, e), jnp.float32)),
        scratch_types=[pltpu.VMEM((2, _CH), jnp.int32),
                       pltpu.VMEM((2, _CH), jnp.int32),
                       pltpu.VMEM((2, _CH, h2), jnp.float32),
                       pltpu.VMEM((2, _CH, h2), jnp.float32),
                       pltpu.VMEM((2, 8, _CH), jnp.float32),
                       pltpu.VMEM((4 * n,), jnp.float32),
                       pltpu.SemaphoreType.DMA,
                       pltpu.SemaphoreType.DMA,
                       pltpu.SemaphoreType.DMA],
    )
    def k(td_h, ts_h, pv_h, src_h, dst_h, s_o, geo_o,
          idxs, idxd, bufd, bufs, gbuf, pvt, semg, semo, semi):
        wid = lax.axis_index("s") * _NC + lax.axis_index("c")
        pltpu.sync_copy(pv_h, pvt)

        def fire_idx(t, b):
            base = (wid + t * _NW) * _CH
            pltpu.async_copy(src_h.at[pl.ds(base, _CH)], idxs.at[b], semi)
            pltpu.async_copy(dst_h.at[pl.ds(base, _CH)], idxd.at[b], semi)

        def wait_idx(t, b):
            base = (wid + t * _NW) * _CH
            pltpu.make_async_copy(
                src_h.at[pl.ds(base, _CH)], idxs.at[b], semi).wait()
            pltpu.make_async_copy(
                dst_h.at[pl.ds(base, _CH)], idxd.at[b], semi).wait()

        def fire_rows(b):
            pltpu.async_copy(td_h.at[idxd.at[b]], bufd.at[b], semg)
            pltpu.async_copy(ts_h.at[idxs.at[b]], bufs.at[b], semg)

        def fire(t, b):
            fire_idx(t, b)
            wait_idx(t, b)
            fire_rows(b)

        def compute(b):
            @plsc.parallel_loop(0, _CH // 16)
            def _(g):
                sl = pl.ds(g * 16, 16)
                s4 = idxs[b, sl] * 4
                d4 = idxd[b, sl] * 4
                rpx = plsc.load_gather(pvt, [s4]) - plsc.load_gather(pvt, [d4])
                rpy = (plsc.load_gather(pvt, [s4 + 1])
                       - plsc.load_gather(pvt, [d4 + 1]))
                rvx = (plsc.load_gather(pvt, [s4 + 2])
                       - plsc.load_gather(pvt, [d4 + 2]))
                rvy = (plsc.load_gather(pvt, [s4 + 3])
                       - plsc.load_gather(pvt, [d4 + 3]))
                dsq = rpx * rpx + rpy * rpy
                dvr = rvx * rpx + rvy * rpy
                r2 = jnp.minimum(1.0 / (dsq + 0.05), 20.0)
                r6 = jnp.minimum(r2 * r2 * r2, 400.0)
                r12 = jnp.minimum(r6 * r6, 160000.0)
                gbuf[b, 0, sl] = dsq
                gbuf[b, 1, sl] = dvr
                gbuf[b, 2, sl] = r2
                gbuf[b, 3, sl] = r6
                gbuf[b, 4, sl] = r12
                gbuf[b, 5, sl] = rpx
                gbuf[b, 6, sl] = rpy
                gbuf[b, 7, sl] = rvx

            @plsc.parallel_loop(0, _CH, unroll=4)
            def _(r):
                for cc in range(h2 // 16):
                    sl = pl.ds(cc * 16, 16)
                    bufd[b, r, sl] = bufd[b, r, sl] + bufs[b, r, sl]

        def drain_out(b, base):
            pltpu.make_async_copy(
                bufd.at[b], s_o.at[pl.ds(base, _CH)], semo).wait()
            pltpu.make_async_copy(
                gbuf.at[b], geo_o.at[:, pl.ds(base, _CH)], semo).wait()

        fire(0, 0)
        fire_idx(1, 1)

        def pair(p, carry):
            for b in range(2):
                t = 2 * p + b
                base = (wid + t * _NW) * _CH

                @pl.when(t >= 2)
                def _():
                    drain_out(b, base)

                @pl.when(t + 1 < ntw)
                def _():
                    wait_idx(t + 1, 1 - b)
                    fire_rows(1 - b)

                pltpu.make_async_copy(
                    td_h.at[idxd.at[b]], bufd.at[b], semg).wait()
                pltpu.make_async_copy(
                    ts_h.at[idxs.at[b]], bufs.at[b], semg).wait()
                compute(b)

                @pl.when(t + 2 < ntw)
                def _():
                    fire_idx(t + 2, b)

                pltpu.async_copy(bufd.at[b], s_o.at[pl.ds(base, _CH)], semo)
                pltpu.async_copy(gbuf.at[b], geo_o.at[:, pl.ds(base, _CH)],
                                 semo)
            return carry

        lax.fori_loop(0, ntw // 2, pair, 0)
        for b in range(2):
            drain_out(b, wid * _CH)

        @pl.when(wid < nextra)
        def _():
            base = (wid + ntw * _NW) * _CH
            fire(ntw, 0)
            pltpu.make_async_copy(
                td_h.at[idxd.at[0]], bufd.at[0], semg).wait()
            pltpu.make_async_copy(
                ts_h.at[idxs.at[0]], bufs.at[0], semg).wait()
            compute(0)
            pltpu.sync_copy(bufd.at[0], s_o.at[pl.ds(base, _CH)])
            pltpu.sync_copy(gbuf.at[0], geo_o.at[:, pl.ds(base, _CH)])

    return k(td, ts, pvflat, src, dst)


def _sp(x):
    return jnp.maximum(x, 0.0) + jnp.log1p(jnp.exp(-jnp.abs(x)))


def _ln(t, g, b):
    mu = jnp.mean(t, axis=1, keepdims=True)
    var = jnp.mean((t - mu) ** 2, axis=1, keepdims=True)
    return (t - mu) * lax.rsqrt(var + 1e-5) * g + b


def _edge_body(s_ref, geo_ref, wg_ref, bcat_ref, mm_ref, lncg_ref, lncb_ref,
               w1c_ref, b1c_ref, w2c_ref, b2c_ref, m_ref):
    s = s_ref[...]
    hh = s.shape[1] // 2
    be = s.shape[0]
    geo = jnp.transpose(geo_ref[...])
    rel_pos = geo[:, 5:7]
    u = s + bcat_ref[...] + jnp.dot(geo, wg_ref[...],
                                    preferred_element_type=jnp.float32)
    mm = mm_ref[...]
    mu = jnp.dot(u, mm, preferred_element_type=jnp.float32)
    du = u - mu
    var = jnp.dot(du * du, mm, preferred_element_type=jnp.float32)
    a1 = _sp(du * lax.rsqrt(var + 1e-5) * lncg_ref[...] + lncb_ref[...])
    t = (jnp.dot(a1, w1c_ref[...], preferred_element_type=jnp.float32)
         + b1c_ref[...])
    w_edge = t[:, hh:hh + 1]
    m_h = (jnp.dot(_sp(t), w2c_ref[...], preferred_element_type=jnp.float32)
           + b2c_ref[...])
    m_v = w_edge * rel_pos
    m_ref[...] = m_h + jnp.concatenate(
        [jnp.zeros((be, hh), jnp.float32), m_v,
         jnp.ones((be, 1), jnp.float32),
         jnp.zeros((be, _MW - hh - 3), jnp.float32)], axis=1)


def _edge(s, geo, wg, bcat, mm, lncg, lncb, w1c, b1c, w2c, b2c):
    e, h2 = s.shape
    be = 2560
    grid = (e // be,)
    full = lambda a: pl.BlockSpec(a.shape, lambda i: (0,) * a.ndim)
    ws = [wg, bcat, mm, lncg, lncb, w1c, b1c, w2c, b2c]
    return pl.pallas_call(
        _edge_body,
        grid=grid,
        in_specs=[pl.BlockSpec((be, h2), lambda i: (i, 0)),
                  pl.BlockSpec((8, be), lambda i: (0, i))] +
                 [full(a) for a in ws],
        out_specs=pl.BlockSpec((be, _MW), lambda i: (i, 0)),
        out_shape=jax.ShapeDtypeStruct((e, _MW), jnp.float32),
    )(s, geo, *ws)


def _sc_scatter(ms, dst, n):
    e = dst.shape[0]
    ncht = e // _CH
    rows_t = (n // (8 * _NS)) * 8
    extra = n - _NS * rows_t
    zch = 104
    nz = rows_t // zch
    assert nz * zch == rows_t and extra <= zch

    qch = [m.shape[0] // _CH for m in ms]
    qoff = [sum(qch[:i]) for i in range(len(ms))]

    @functools.partial(
        pl.kernel, mesh=_mesh(),
        out_type=jax.ShapeDtypeStruct((_NC, n, _MW), jnp.float32),
        scratch_types=[pltpu.VMEM((2, _CH), jnp.int32),
                       pltpu.VMEM((2, _CH, _MW), jnp.float32),
                       pltpu.VMEM((zch, _MW), jnp.float32),
                       pltpu.VMEM_SHARED((n, _MW), jnp.float32),
                       pltpu.SemaphoreType.DMA],
    )
    def k(*refs):
        m_hs = refs[:len(ms)]
        dst_h, p_o, idxd, mbuf, zbuf, acc, semf = refs[len(ms):]
        cid = lax.axis_index("c")
        sid = lax.axis_index("s")
        wid = sid * _NC + cid

        def zrow(r, carry):
            for cc in range(_MW // 16):
                zbuf[r, pl.ds(cc * 16, 16)] = jnp.zeros((16,), jnp.float32)
            return carry

        lax.fori_loop(0, zch, zrow, 0)
        for j in range(nz):
            pltpu.sync_copy(zbuf, acc.at[pl.ds(sid * rows_t + j * zch, zch)])

        @pl.when(sid == 0)
        def _():
            pltpu.sync_copy(zbuf.at[pl.ds(0, extra)],
                            acc.at[pl.ds(_NS * rows_t, extra)])

        plsc.subcore_barrier()

        for q, m_h in enumerate(m_hs):
            ncq = qch[q]
            nte = (ncq // (2 * _NW)) * (2 * _NW)
            ntw = nte // _NW
            nextra = ncq - nte
            goff = qoff[q] * _CH

            def fire(t, b):
                base = (wid + t * _NW) * _CH
                pltpu.async_copy(dst_h.at[pl.ds(goff + base, _CH)],
                                 idxd.at[b], semf)
                pltpu.async_copy(m_h.at[pl.ds(base, _CH)], mbuf.at[b], semf)

            def wait_fire(t, b):
                base = (wid + t * _NW) * _CH
                pltpu.make_async_copy(
                    dst_h.at[pl.ds(goff + base, _CH)], idxd.at[b],
                    semf).wait()
                pltpu.make_async_copy(
                    m_h.at[pl.ds(base, _CH)], mbuf.at[b], semf).wait()

            fire(0, 0)

            def pair(p, carry):
                for b in range(2):
                    t = 2 * p + b

                    @pl.when(t + 1 < ntw)
                    def _():
                        fire(t + 1, 1 - b)

                    wait_fire(t, b)
                    pltpu.sync_copy(mbuf.at[b], acc.at[idxd.at[b]], add=True)
                return carry

            lax.fori_loop(0, ntw // 2, pair, 0)

            @pl.when(wid < nextra)
            def _():
                fire(ntw, 0)
                wait_fire(ntw, 0)
                pltpu.sync_copy(mbuf.at[0], acc.at[idxd.at[0]], add=True)

        plsc.subcore_barrier()
        pltpu.sync_copy(acc.at[pl.ds(sid * rows_t, rows_t)],
                        p_o.at[cid, pl.ds(sid * rows_t, rows_t)])

        @pl.when(sid == 0)
        def _():
            pltpu.sync_copy(acc.at[pl.ds(_NS * rows_t, extra)],
                            p_o.at[cid, pl.ds(_NS * rows_t, extra)])

    return k(*ms, dst)


def _dec_body(x_ref, p_ref, pb_ref, w0x_ref, w0a_ref, w0n_ref, hb0_ref,
              hlng_ref, hlnb_ref, hw1_ref, hb1_ref, o_ref):
    xv = x_ref[...]
    hh = w0a_ref.shape[0]
    p = (p_ref[0] + p_ref[1]) + (pb_ref[0] + pb_ref[1])
    cnt = jnp.maximum(p[:, hh + 2:hh + 3], 1.0)
    agg_h = p[:, 0:hh] / cnt
    agg_v = p[:, hh:hh + 2] / cnt + 1e-8
    mvn = jnp.sqrt(jnp.sum(agg_v * agg_v, axis=1, keepdims=True))
    t = (jnp.dot(xv, w0x_ref[...], preferred_element_type=jnp.float32)
         + jnp.dot(agg_h, w0a_ref[...], preferred_element_type=jnp.float32)
         + mvn * w0n_ref[...] + hb0_ref[...])
    t = jax.nn.softplus(_ln(t, hlng_ref[...], hlnb_ref[...]))
    o_ref[...] = (xv
                  + jnp.dot(t, hw1_ref[...], preferred_element_type=jnp.float32)
                  + hb1_ref[...])


def _decode(x, pa, pb, w0x, w0a, w0n, hb0, hlng, hlnb, hw1, hb1):
    n, d = x.shape
    bn = 2000
    grid = (n // bn,)
    full = lambda a: pl.BlockSpec(a.shape, lambda i: (0,) * a.ndim)
    ws = [w0x, w0a, w0n, hb0, hlng, hlnb, hw1, hb1]
    return pl.pallas_call(
        _dec_body,
        grid=grid,
        in_specs=[pl.BlockSpec((bn, d), lambda i: (i, 0)),
                  pl.BlockSpec((_NC, bn, _MW), lambda i: (0, i, 0)),
                  pl.BlockSpec((_NC, bn, _MW), lambda i: (0, i, 0))] +
                 [full(a) for a in ws],
        out_specs=pl.BlockSpec((bn, d), lambda i: (i, 0)),
        out_shape=jax.ShapeDtypeStruct((n, d), jnp.float32),
    )(x, pa, pb, *ws)


def kernel(x, pos, vel, edge_index,
           e_w0, e_b0, e_ln_g, e_ln_b, e_w1, e_b1, e_w2, e_b2,
           v_w0, v_b0, v_ln_g, v_ln_b, v_w1, v_b1,
           h_w0, h_b0, h_ln_g, h_ln_b, h_w1, h_b1):
    n, d = x.shape
    hh = e_b0.shape[0]
    src = edge_index[0].astype(jnp.int32)
    dst = edge_index[1].astype(jnp.int32)

    wd = jnp.concatenate([e_w0[:d], v_w0[:d]], axis=1)
    ws_ = jnp.concatenate([e_w0[d:2 * d], v_w0[d:2 * d]], axis=1)
    wg = jnp.concatenate([e_w0[2 * d:], v_w0[2 * d:]], axis=1)
    wg = jnp.concatenate([wg, jnp.zeros((3, 2 * hh), jnp.float32)], axis=0)
    bcat = jnp.concatenate([e_b0, v_b0]).reshape(1, 2 * hh)
    pvflat = jnp.concatenate([pos, vel], axis=1).reshape(-1)

    h2 = 2 * hh
    zb = jnp.zeros((hh, hh), jnp.float32)
    jb = jnp.full((hh, hh), 1.0 / hh, jnp.float32)
    mm = jnp.concatenate([jnp.concatenate([jb, zb], 1),
                          jnp.concatenate([zb, jb], 1)], 0)
    lncg = jnp.concatenate([e_ln_g, v_ln_g]).reshape(1, h2)
    lncb = jnp.concatenate([e_ln_b, v_ln_b]).reshape(1, h2)
    vw1p = jnp.concatenate([v_w1, jnp.zeros((hh, hh - 1), jnp.float32)], 1)
    w1c = jnp.concatenate([jnp.concatenate([e_w1, zb], 1),
                           jnp.concatenate([zb, vw1p], 1)], 0)
    b1c = jnp.concatenate(
        [e_b1, v_b1, jnp.zeros((hh - 1,), jnp.float32)]).reshape(1, h2)
    w2c = jnp.concatenate([jnp.concatenate([e_w2, zb], 1),
                           jnp.concatenate([zb, zb], 1)], 0)
    b2c = jnp.concatenate([e_b2, jnp.zeros((hh,), jnp.float32)]).reshape(1, h2)

    td, ts = _prep(x, wd, ws_)
    e = src.shape[0]
    ncht = e // _CH
    sl_ch = [640, 640, 640, ncht - 1920]
    nspl = 2
    bounds = [0]
    for c in sl_ch:
        bounds.append(bounds[-1] + c * _CH)
    ms = []
    for q in range(len(sl_ch)):
        lo, hi = bounds[q], bounds[q + 1]
        s_q, geo_q = _sc_gather(td, ts, pvflat, src[lo:hi], dst[lo:hi])
        ms.append(_edge(s_q, geo_q, wg, bcat, mm, lncg, lncb,
                        w1c, b1c, w2c, b2c))
    mid = bounds[nspl]
    pa = _sc_scatter(ms[:nspl], dst[:mid], n)
    pb = _sc_scatter(ms[nspl:], dst[mid:], n)
    out = _decode(x, pa, pb, h_w0[:d], h_w0[d:d + hh],
                  h_w0[d + hh:].reshape(1, hh),
                  h_b0.reshape(1, hh), h_ln_g.reshape(1, hh),
                  h_ln_b.reshape(1, hh), h_w1, h_b1.reshape(1, d))
    return out


# speedup vs baseline: 1.0900x; 1.0513x over previous
import functools

import jax
import jax.numpy as jnp
from jax import lax
from jax.experimental import pallas as pl
from jax.experimental.pallas import tpu as pltpu
from jax.experimental.pallas import tpu_sc as plsc

_NC = 2
_NS = 16
_NW = _NC * _NS
_CH = 128
_MW = 128


def _mesh():
    return plsc.VectorSubcoreMesh(
        core_axis_name="c", subcore_axis_name="s",
        num_cores=_NC, num_subcores=_NS)


def _prep_body(x_ref, wd_ref, ws_ref, td_ref, ts_ref):
    xv = x_ref[...]
    td_ref[...] = jnp.dot(xv, wd_ref[...], preferred_element_type=jnp.float32)
    ts_ref[...] = jnp.dot(xv, ws_ref[...], preferred_element_type=jnp.float32)


def _prep(x, wd, ws):
    n = x.shape[0]
    h2 = wd.shape[1]
    return pl.pallas_call(
        _prep_body,
        out_shape=(jax.ShapeDtypeStruct((n, h2), jnp.float32),
                   jax.ShapeDtypeStruct((n, h2), jnp.float32)),
    )(x, wd, ws)


def _sc_gather(td, ts, pvflat, src, dst):
    n, h2 = td.shape
    e = src.shape[0]
    ncht = e // _CH

    nte = (ncht // (2 * _NW)) * (2 * _NW)
    ntw = nte // _NW
    nextra = ncht - nte

    @functools.partial(
        pl.kernel, mesh=_mesh(),
        compiler_params=pltpu.CompilerParams(needs_layout_passes=False),
        out_type=(jax.ShapeDtypeStruct((e, h2), jnp.float32),
                  jax.ShapeDtypeStruct((8, e), jnp.float32)),
        scratch_types=[pltpu.VMEM((2, _CH), jnp.int32),
                       pltpu.VMEM((2, _CH), jnp.int32),
                       pltpu.VMEM((2, _CH, h2), jnp.float32),
                       pltpu.VMEM((2, _CH, h2), jnp.float32),
                       pltpu.VMEM((2, 8, _CH), jnp.float32),
                       pltpu.VMEM((4 * n,), jnp.float32),
                       pltpu.SemaphoreType.DMA,
                       pltpu.SemaphoreType.DMA,
                       pltpu.SemaphoreType.DMA],
    )
    def k(td_h, ts_h, pv_h, src_h, dst_h, s_o, geo_o,
          idxs, idxd, bufd, bufs, gbuf, pvt, semg, semo, semi):
        wid = lax.axis_index("s") * _NC + lax.axis_index("c")
        pltpu.sync_copy(pv_h, pvt)

        def fire_idx(t, b):
            base = (wid + t * _NW) * _CH
            pltpu.async_copy(src_h.at[pl.ds(base, _CH)], idxs.at[b], semi)
            pltpu.async_copy(dst_h.at[pl.ds(base, _CH)], idxd.at[b], semi)

        def wait_idx(t, b):
            base = (wid + t * _NW) * _CH
            pltpu.make_async_copy(
                src_h.at[pl.ds(base, _CH)], idxs.at[b], semi).wait()
            pltpu.make_async_copy(
                dst_h.at[pl.ds(base, _CH)], idxd.at[b], semi).wait()

        def fire_rows(b):
            pltpu.async_copy(td_h.at[idxd.at[b]], bufd.at[b], semg)
            pltpu.async_copy(ts_h.at[idxs.at[b]], bufs.at[b], semg)

        def fire(t, b):
            fire_idx(t, b)
            wait_idx(t, b)
            fire_rows(b)

        def compute(b):
            @plsc.parallel_loop(0, _CH // 16)
            def _(g):
                sl = pl.ds(g * 16, 16)
                s4 = idxs[b, sl] * 4
                d4 = idxd[b, sl] * 4
                rpx = plsc.load_gather(pvt, [s4]) - plsc.load_gather(pvt, [d4])
                rpy = (plsc.load_gather(pvt, [s4 + 1])
                       - plsc.load_gather(pvt, [d4 + 1]))
                rvx = (plsc.load_gather(pvt, [s4 + 2])
                       - plsc.load_gather(pvt, [d4 + 2]))
                rvy = (plsc.load_gather(pvt, [s4 + 3])
                       - plsc.load_gather(pvt, [d4 + 3]))
                dsq = rpx * rpx + rpy * rpy
                dvr = rvx * rpx + rvy * rpy
                r2 = jnp.minimum(1.0 / (dsq + 0.05), 20.0)
                r6 = jnp.minimum(r2 * r2 * r2, 400.0)
                r12 = jnp.minimum(r6 * r6, 160000.0)
                gbuf[b, 0, sl] = dsq
                gbuf[b, 1, sl] = dvr
                gbuf[b, 2, sl] = r2
                gbuf[b, 3, sl] = r6
                gbuf[b, 4, sl] = r12
                gbuf[b, 5, sl] = rpx
                gbuf[b, 6, sl] = rpy
                gbuf[b, 7, sl] = rvx

            @plsc.parallel_loop(0, _CH, unroll=4)
            def _(r):
                for cc in range(h2 // 16):
                    sl = pl.ds(cc * 16, 16)
                    bufd[b, r, sl] = bufd[b, r, sl] + bufs[b, r, sl]

        def drain_out(b, base):
            pltpu.make_async_copy(
                bufd.at[b], s_o.at[pl.ds(base, _CH)], semo).wait()
            pltpu.make_async_copy(
                gbuf.at[b], geo_o.at[:, pl.ds(base, _CH)], semo).wait()

        fire(0, 0)
        fire_idx(1, 1)

        def pair(p, carry):
            for b in range(2):
                t = 2 * p + b
                base = (wid + t * _NW) * _CH

                @pl.when(t >= 2)
                def _():
                    drain_out(b, base)

                @pl.when(t + 1 < ntw)
                def _():
                    wait_idx(t + 1, 1 - b)
                    fire_rows(1 - b)

                pltpu.make_async_copy(
                    td_h.at[idxd.at[b]], bufd.at[b], semg).wait()
                pltpu.make_async_copy(
                    ts_h.at[idxs.at[b]], bufs.at[b], semg).wait()
                compute(b)

                @pl.when(t + 2 < ntw)
                def _():
                    fire_idx(t + 2, b)

                pltpu.async_copy(bufd.at[b], s_o.at[pl.ds(base, _CH)], semo)
                pltpu.async_copy(gbuf.at[b], geo_o.at[:, pl.ds(base, _CH)],
                                 semo)
            return carry

        lax.fori_loop(0, ntw // 2, pair, 0)
        for b in range(2):
            drain_out(b, wid * _CH)

        @pl.when(wid < nextra)
        def _():
            base = (wid + ntw * _NW) * _CH
            fire(ntw, 0)
            pltpu.make_async_copy(
                td_h.at[idxd.at[0]], bufd.at[0], semg).wait()
            pltpu.make_async_copy(
                ts_h.at[idxs.at[0]], bufs.at[0], semg).wait()
            compute(0)
            pltpu.sync_copy(bufd.at[0], s_o.at[pl.ds(base, _CH)])
            pltpu.sync_copy(gbuf.at[0], geo_o.at[:, pl.ds(base, _CH)])

    return k(td, ts, pvflat, src, dst)


def _sp(x):
    return jnp.maximum(x, 0.0) + jnp.log1p(jnp.exp(-jnp.abs(x)))


def _ln(t, g, b):
    mu = jnp.mean(t, axis=1, keepdims=True)
    var = jnp.mean((t - mu) ** 2, axis=1, keepdims=True)
    return (t - mu) * lax.rsqrt(var + 1e-5) * g + b


def _edge_body(s_ref, geo_ref, wg_ref, bcat_ref, mm_ref, lncg_ref, lncb_ref,
               w1c_ref, b1c_ref, w2c_ref, b2c_ref, rpsel_ref, m_ref):
    s = s_ref[...]
    hh = s.shape[1] // 2
    geo = geo_ref[...]
    dn = (((0,), (0,)), ((), ()))
    u = s + bcat_ref[...] + lax.dot_general(
        geo, wg_ref[...], dn, preferred_element_type=jnp.float32)
    mm = mm_ref[...]
    mu = jnp.dot(u, mm, preferred_element_type=jnp.float32)
    du = u - mu
    var = jnp.dot(du * du, mm, preferred_element_type=jnp.float32)
    a1 = _sp(du * lax.rsqrt(var + 1e-5) * lncg_ref[...] + lncb_ref[...])
    t = (jnp.dot(a1, w1c_ref[...], preferred_element_type=jnp.float32)
         + b1c_ref[...])
    w_edge = t[:, hh:hh + 1]
    m_h = (jnp.dot(_sp(t), w2c_ref[...], preferred_element_type=jnp.float32)
           + b2c_ref[...])
    rp_pad = lax.dot_general(geo, rpsel_ref[...], dn,
                             preferred_element_type=jnp.float32)
    m_ref[...] = m_h + w_edge * rp_pad


def _edge(s, geo, wg, bcat, mm, lncg, lncb, w1c, b1c, w2c, b2c, rpsel):
    e, h2 = s.shape
    be = 2560
    grid = (e // be,)
    full = lambda a: pl.BlockSpec(a.shape, lambda i: (0,) * a.ndim)
    ws = [wg, bcat, mm, lncg, lncb, w1c, b1c, w2c, b2c, rpsel]
    return pl.pallas_call(
        _edge_body,
        grid=grid,
        in_specs=[pl.BlockSpec((be, h2), lambda i: (i, 0)),
                  pl.BlockSpec((8, be), lambda i: (0, i))] +
                 [full(a) for a in ws],
        out_specs=pl.BlockSpec((be, _MW), lambda i: (i, 0)),
        out_shape=jax.ShapeDtypeStruct((e, _MW), jnp.float32),
    )(s, geo, *ws)


def _sc_scatter(ms, dst, n):
    e = dst.shape[0]
    ncht = e // _CH
    rows_t = (n // (8 * _NS)) * 8
    extra = n - _NS * rows_t
    zch = 104
    nz = rows_t // zch
    assert nz * zch == rows_t and extra <= zch

    qch = [m.shape[0] // _CH for m in ms]
    qoff = [sum(qch[:i]) for i in range(len(ms))]

    @functools.partial(
        pl.kernel, mesh=_mesh(),
        out_type=jax.ShapeDtypeStruct((_NC, n, _MW), jnp.float32),
        scratch_types=[pltpu.VMEM((2, _CH), jnp.int32),
                       pltpu.VMEM((2, _CH, _MW), jnp.float32),
                       pltpu.VMEM((zch, _MW), jnp.float32),
                       pltpu.VMEM_SHARED((n, _MW), jnp.float32),
                       pltpu.SemaphoreType.DMA],
    )
    def k(*refs):
        m_hs = refs[:len(ms)]
        dst_h, p_o, idxd, mbuf, zbuf, acc, semf = refs[len(ms):]
        cid = lax.axis_index("c")
        sid = lax.axis_index("s")
        wid = sid * _NC + cid

        def zrow(r, carry):
            for cc in range(_MW // 16):
                zbuf[r, pl.ds(cc * 16, 16)] = jnp.zeros((16,), jnp.float32)
            return carry

        lax.fori_loop(0, zch, zrow, 0)
        for j in range(nz):
            pltpu.sync_copy(zbuf, acc.at[pl.ds(sid * rows_t + j * zch, zch)])

        @pl.when(sid == 0)
        def _():
            pltpu.sync_copy(zbuf.at[pl.ds(0, extra)],
                            acc.at[pl.ds(_NS * rows_t, extra)])

        plsc.subcore_barrier()

        for q, m_h in enumerate(m_hs):
            ncq = qch[q]
            nte = (ncq // (2 * _NW)) * (2 * _NW)
            ntw = nte // _NW
            nextra = ncq - nte
            goff = qoff[q] * _CH

            def fire(t, b):
                base = (wid + t * _NW) * _CH
                pltpu.async_copy(dst_h.at[pl.ds(goff + base, _CH)],
                                 idxd.at[b], semf)
                pltpu.async_copy(m_h.at[pl.ds(base, _CH)], mbuf.at[b], semf)

            def wait_fire(t, b):
                base = (wid + t * _NW) * _CH
                pltpu.make_async_copy(
                    dst_h.at[pl.ds(goff + base, _CH)], idxd.at[b],
                    semf).wait()
                pltpu.make_async_copy(
                    m_h.at[pl.ds(base, _CH)], mbuf.at[b], semf).wait()

            fire(0, 0)

            def pair(p, carry):
                for b in range(2):
                    t = 2 * p + b

                    @pl.when(t + 1 < ntw)
                    def _():
                        fire(t + 1, 1 - b)

                    wait_fire(t, b)
                    pltpu.sync_copy(mbuf.at[b], acc.at[idxd.at[b]], add=True)
                return carry

            lax.fori_loop(0, ntw // 2, pair, 0)

            @pl.when(wid < nextra)
            def _():
                fire(ntw, 0)
                wait_fire(ntw, 0)
                pltpu.sync_copy(mbuf.at[0], acc.at[idxd.at[0]], add=True)

        plsc.subcore_barrier()
        pltpu.sync_copy(acc.at[pl.ds(sid * rows_t, rows_t)],
                        p_o.at[cid, pl.ds(sid * rows_t, rows_t)])

        @pl.when(sid == 0)
        def _():
            pltpu.sync_copy(acc.at[pl.ds(_NS * rows_t, extra)],
                            p_o.at[cid, pl.ds(_NS * rows_t, extra)])

    return k(*ms, dst)


def _dec_body(x_ref, p_ref, pb_ref, w0x_ref, w0a_ref, w0n_ref, hb0_ref,
              hlng_ref, hlnb_ref, hw1_ref, hb1_ref, o_ref):
    xv = x_ref[...]
    hh = w0a_ref.shape[0]
    p = (p_ref[0] + p_ref[1]) + (pb_ref[0] + pb_ref[1])
    cnt = jnp.maximum(p[:, hh + 2:hh + 3], 1.0)
    agg_h = p[:, 0:hh] / cnt
    agg_v = p[:, hh:hh + 2] / cnt + 1e-8
    mvn = jnp.sqrt(jnp.sum(agg_v * agg_v, axis=1, keepdims=True))
    t = (jnp.dot(xv, w0x_ref[...], preferred_element_type=jnp.float32)
         + jnp.dot(agg_h, w0a_ref[...], preferred_element_type=jnp.float32)
         + mvn * w0n_ref[...] + hb0_ref[...])
    t = jax.nn.softplus(_ln(t, hlng_ref[...], hlnb_ref[...]))
    o_ref[...] = (xv
                  + jnp.dot(t, hw1_ref[...], preferred_element_type=jnp.float32)
                  + hb1_ref[...])


def _decode(x, pa, pb, w0x, w0a, w0n, hb0, hlng, hlnb, hw1, hb1):
    n, d = x.shape
    bn = 2000
    grid = (n // bn,)
    full = lambda a: pl.BlockSpec(a.shape, lambda i: (0,) * a.ndim)
    ws = [w0x, w0a, w0n, hb0, hlng, hlnb, hw1, hb1]
    return pl.pallas_call(
        _dec_body,
        grid=grid,
        in_specs=[pl.BlockSpec((bn, d), lambda i: (i, 0)),
                  pl.BlockSpec((_NC, bn, _MW), lambda i: (0, i, 0)),
                  pl.BlockSpec((_NC, bn, _MW), lambda i: (0, i, 0))] +
                 [full(a) for a in ws],
        out_specs=pl.BlockSpec((bn, d), lambda i: (i, 0)),
        out_shape=jax.ShapeDtypeStruct((n, d), jnp.float32),
    )(x, pa, pb, *ws)


def kernel(x, pos, vel, edge_index,
           e_w0, e_b0, e_ln_g, e_ln_b, e_w1, e_b1, e_w2, e_b2,
           v_w0, v_b0, v_ln_g, v_ln_b, v_w1, v_b1,
           h_w0, h_b0, h_ln_g, h_ln_b, h_w1, h_b1):
    n, d = x.shape
    hh = e_b0.shape[0]
    src = edge_index[0].astype(jnp.int32)
    dst = edge_index[1].astype(jnp.int32)

    wd = jnp.concatenate([e_w0[:d], v_w0[:d]], axis=1)
    ws_ = jnp.concatenate([e_w0[d:2 * d], v_w0[d:2 * d]], axis=1)
    wg = jnp.concatenate([e_w0[2 * d:], v_w0[2 * d:]], axis=1)
    wg = jnp.concatenate([wg, jnp.zeros((3, 2 * hh), jnp.float32)], axis=0)
    bcat = jnp.concatenate([e_b0, v_b0]).reshape(1, 2 * hh)
    pvflat = jnp.concatenate([pos, vel], axis=1).reshape(-1)

    h2 = 2 * hh
    zb = jnp.zeros((hh, hh), jnp.float32)
    jb = jnp.full((hh, hh), 1.0 / hh, jnp.float32)
    mm = jnp.concatenate([jnp.concatenate([jb, zb], 1),
                          jnp.concatenate([zb, jb], 1)], 0)
    lncg = jnp.concatenate([e_ln_g, v_ln_g]).reshape(1, h2)
    lncb = jnp.concatenate([e_ln_b, v_ln_b]).reshape(1, h2)
    vw1p = jnp.concatenate([v_w1, jnp.zeros((hh, hh - 1), jnp.float32)], 1)
    w1c = jnp.concatenate([jnp.concatenate([e_w1, zb], 1),
                           jnp.concatenate([zb, vw1p], 1)], 0)
    b1c = jnp.concatenate(
        [e_b1, v_b1, jnp.zeros((hh - 1,), jnp.float32)]).reshape(1, h2)
    w2c = jnp.concatenate([jnp.concatenate([e_w2, zb], 1),
                           jnp.concatenate([zb, zb], 1)], 0)
    b2c = (jnp.concatenate([e_b2, jnp.zeros((hh,), jnp.float32)])
           .at[hh + 2].set(1.0).reshape(1, h2))
    rpsel = (jnp.zeros((8, h2), jnp.float32)
             .at[5, hh].set(1.0).at[6, hh + 1].set(1.0))

    td, ts = _prep(x, wd, ws_)
    e = src.shape[0]
    ncht = e // _CH
    sl_ch = [640, 640, 640, ncht - 1920]
    nspl = 2
    bounds = [0]
    for c in sl_ch:
        bounds.append(bounds[-1] + c * _CH)
    ms = []
    for q in range(len(sl_ch)):
        lo, hi = bounds[q], bounds[q + 1]
        s_q, geo_q = _sc_gather(td, ts, pvflat, src[lo:hi], dst[lo:hi])
        ms.append(_edge(s_q, geo_q, wg, bcat, mm, lncg, lncb,
                        w1c, b1c, w2c, b2c, rpsel))
    mid = bounds[nspl]
    pa = _sc_scatter(ms[:nspl], dst[:mid], n)
    pb = _sc_scatter(ms[nspl:], dst[mid:], n)
    out = _decode(x, pa, pb, h_w0[:d], h_w0[d:d + hh],
                  h_w0[d + hh:].reshape(1, hh),
                  h_b0.reshape(1, hh), h_ln_g.reshape(1, hh),
                  h_ln_b.reshape(1, hh), h_w1, h_b1.reshape(1, d))
    return out


# speedup vs baseline: 1.0907x; 1.0007x over previous
import functools

import jax
import jax.numpy as jnp
from jax import lax
from jax.experimental import pallas as pl
from jax.experimental.pallas import tpu as pltpu
from jax.experimental.pallas import tpu_sc as plsc

_NC = 2
_NS = 16
_NW = _NC * _NS
_CH = 128
_MW = 128


def _mesh():
    return plsc.VectorSubcoreMesh(
        core_axis_name="c", subcore_axis_name="s",
        num_cores=_NC, num_subcores=_NS)


def _prep_body(x_ref, wd_ref, ws_ref, td_ref, ts_ref):
    xv = x_ref[...]
    td_ref[...] = jnp.dot(xv, wd_ref[...], preferred_element_type=jnp.float32)
    ts_ref[...] = jnp.dot(xv, ws_ref[...], preferred_element_type=jnp.float32)


def _prep(x, wd, ws):
    n = x.shape[0]
    h2 = wd.shape[1]
    return pl.pallas_call(
        _prep_body,
        out_shape=(jax.ShapeDtypeStruct((n, h2), jnp.float32),
                   jax.ShapeDtypeStruct((n, h2), jnp.float32)),
    )(x, wd, ws)


def _sc_gather(td, ts, pvflat, src, dst):
    n, h2 = td.shape
    e = src.shape[0]
    ncht = e // _CH

    nte = (ncht // (2 * _NW)) * (2 * _NW)
    ntw = nte // _NW
    nextra = ncht - nte

    @functools.partial(
        pl.kernel, mesh=_mesh(),
        compiler_params=pltpu.CompilerParams(needs_layout_passes=False),
        out_type=(jax.ShapeDtypeStruct((e, h2), jnp.float32),
                  jax.ShapeDtypeStruct((8, e), jnp.float32)),
        scratch_types=[pltpu.VMEM((2, _CH), jnp.int32),
                       pltpu.VMEM((2, _CH), jnp.int32),
                       pltpu.VMEM((2, _CH, h2), jnp.float32),
                       pltpu.VMEM((2, _CH, h2), jnp.float32),
                       pltpu.VMEM((2, 8, _CH), jnp.float32),
                       pltpu.VMEM((4 * n,), jnp.float32),
                       pltpu.SemaphoreType.DMA,
                       pltpu.SemaphoreType.DMA,
                       pltpu.SemaphoreType.DMA],
    )
    def k(td_h, ts_h, pv_h, src_h, dst_h, s_o, geo_o,
          idxs, idxd, bufd, bufs, gbuf, pvt, semg, semo, semi):
        wid = lax.axis_index("s") * _NC + lax.axis_index("c")
        pltpu.sync_copy(pv_h, pvt)

        def fire_idx(t, b):
            base = (wid + t * _NW) * _CH
            pltpu.async_copy(src_h.at[pl.ds(base, _CH)], idxs.at[b], semi)
            pltpu.async_copy(dst_h.at[pl.ds(base, _CH)], idxd.at[b], semi)

        def wait_idx(t, b):
            base = (wid + t * _NW) * _CH
            pltpu.make_async_copy(
                src_h.at[pl.ds(base, _CH)], idxs.at[b], semi).wait()
            pltpu.make_async_copy(
                dst_h.at[pl.ds(base, _CH)], idxd.at[b], semi).wait()

        def fire_rows(b):
            pltpu.async_copy(td_h.at[idxd.at[b]], bufd.at[b], semg)
            pltpu.async_copy(ts_h.at[idxs.at[b]], bufs.at[b], semg)

        def fire(t, b):
            fire_idx(t, b)
            wait_idx(t, b)
            fire_rows(b)

        def compute(b):
            @plsc.parallel_loop(0, _CH // 16)
            def _(g):
                sl = pl.ds(g * 16, 16)
                s4 = idxs[b, sl] * 4
                d4 = idxd[b, sl] * 4
                rpx = plsc.load_gather(pvt, [s4]) - plsc.load_gather(pvt, [d4])
                rpy = (plsc.load_gather(pvt, [s4 + 1])
                       - plsc.load_gather(pvt, [d4 + 1]))
                rvx = (plsc.load_gather(pvt, [s4 + 2])
                       - plsc.load_gather(pvt, [d4 + 2]))
                rvy = (plsc.load_gather(pvt, [s4 + 3])
                       - plsc.load_gather(pvt, [d4 + 3]))
                dsq = rpx * rpx + rpy * rpy
                dvr = rvx * rpx + rvy * rpy
                r2 = jnp.minimum(1.0 / (dsq + 0.05), 20.0)
                r6 = jnp.minimum(r2 * r2 * r2, 400.0)
                r12 = jnp.minimum(r6 * r6, 160000.0)
                gbuf[b, 0, sl] = dsq
                gbuf[b, 1, sl] = dvr
                gbuf[b, 2, sl] = r2
                gbuf[b, 3, sl] = r6
                gbuf[b, 4, sl] = r12
                gbuf[b, 5, sl] = rpx
                gbuf[b, 6, sl] = rpy
                gbuf[b, 7, sl] = rvx

            @plsc.parallel_loop(0, _CH, unroll=8)
            def _(r):
                for cc in range(h2 // 16):
                    sl = pl.ds(cc * 16, 16)
                    bufd[b, r, sl] = bufd[b, r, sl] + bufs[b, r, sl]

        def drain_out(b, base):
            pltpu.make_async_copy(
                bufd.at[b], s_o.at[pl.ds(base, _CH)], semo).wait()
            pltpu.make_async_copy(
                gbuf.at[b], geo_o.at[:, pl.ds(base, _CH)], semo).wait()

        fire(0, 0)
        fire_idx(1, 1)

        def pair(p, carry):
            for b in range(2):
                t = 2 * p + b
                base = (wid + t * _NW) * _CH

                @pl.when(t >= 2)
                def _():
                    drain_out(b, base)

                @pl.when(t + 1 < ntw)
                def _():
                    wait_idx(t + 1, 1 - b)
                    fire_rows(1 - b)

                pltpu.make_async_copy(
                    td_h.at[idxd.at[b]], bufd.at[b], semg).wait()
                pltpu.make_async_copy(
                    ts_h.at[idxs.at[b]], bufs.at[b], semg).wait()
                compute(b)

                @pl.when(t + 2 < ntw)
                def _():
                    fire_idx(t + 2, b)

                pltpu.async_copy(bufd.at[b], s_o.at[pl.ds(base, _CH)], semo)
                pltpu.async_copy(gbuf.at[b], geo_o.at[:, pl.ds(base, _CH)],
                                 semo)
            return carry

        lax.fori_loop(0, ntw // 2, pair, 0)
        for b in range(2):
            drain_out(b, wid * _CH)

        @pl.when(wid < nextra)
        def _():
            base = (wid + ntw * _NW) * _CH
            fire(ntw, 0)
            pltpu.make_async_copy(
                td_h.at[idxd.at[0]], bufd.at[0], semg).wait()
            pltpu.make_async_copy(
                ts_h.at[idxs.at[0]], bufs.at[0], semg).wait()
            compute(0)
            pltpu.sync_copy(bufd.at[0], s_o.at[pl.ds(base, _CH)])
            pltpu.sync_copy(gbuf.at[0], geo_o.at[:, pl.ds(base, _CH)])

    return k(td, ts, pvflat, src, dst)


def _sp(x):
    return jnp.maximum(x, 0.0) + jnp.log1p(jnp.exp(-jnp.abs(x)))


def _ln(t, g, b):
    mu = jnp.mean(t, axis=1, keepdims=True)
    var = jnp.mean((t - mu) ** 2, axis=1, keepdims=True)
    return (t - mu) * lax.rsqrt(var + 1e-5) * g + b


def _edge_body(s_ref, geo_ref, wg_ref, bcat_ref, mm_ref, lncg_ref, lncb_ref,
               w1c_ref, b1c_ref, w2c_ref, b2c_ref, rpsel_ref, m_ref):
    s = s_ref[...]
    hh = s.shape[1] // 2
    geo = geo_ref[...]
    dn = (((0,), (0,)), ((), ()))
    u = s + bcat_ref[...] + lax.dot_general(
        geo, wg_ref[...], dn, preferred_element_type=jnp.float32)
    mm = mm_ref[...]
    mu = jnp.dot(u, mm, preferred_element_type=jnp.float32)
    du = u - mu
    var = jnp.dot(du * du, mm, preferred_element_type=jnp.float32)
    a1 = _sp(du * lax.rsqrt(var + 1e-5) * lncg_ref[...] + lncb_ref[...])
    t = (jnp.dot(a1, w1c_ref[...], preferred_element_type=jnp.float32)
         + b1c_ref[...])
    w_edge = t[:, hh:hh + 1]
    m_h = (jnp.dot(_sp(t), w2c_ref[...], preferred_element_type=jnp.float32)
           + b2c_ref[...])
    rp_pad = lax.dot_general(geo, rpsel_ref[...], dn,
                             preferred_element_type=jnp.float32)
    m_ref[...] = m_h + w_edge * rp_pad


def _edge(s, geo, wg, bcat, mm, lncg, lncb, w1c, b1c, w2c, b2c, rpsel):
    e, h2 = s.shape
    be = 2560
    grid = (e // be,)
    full = lambda a: pl.BlockSpec(a.shape, lambda i: (0,) * a.ndim)
    ws = [wg, bcat, mm, lncg, lncb, w1c, b1c, w2c, b2c, rpsel]
    return pl.pallas_call(
        _edge_body,
        grid=grid,
        in_specs=[pl.BlockSpec((be, h2), lambda i: (i, 0)),
                  pl.BlockSpec((8, be), lambda i: (0, i))] +
                 [full(a) for a in ws],
        out_specs=pl.BlockSpec((be, _MW), lambda i: (i, 0)),
        out_shape=jax.ShapeDtypeStruct((e, _MW), jnp.float32),
    )(s, geo, *ws)


def _sc_scatter(ms, dst, n):
    e = dst.shape[0]
    ncht = e // _CH
    rows_t = (n // (8 * _NS)) * 8
    extra = n - _NS * rows_t
    zch = 104
    nz = rows_t // zch
    assert nz * zch == rows_t and extra <= zch

    qch = [m.shape[0] // _CH for m in ms]
    qoff = [sum(qch[:i]) for i in range(len(ms))]

    @functools.partial(
        pl.kernel, mesh=_mesh(),
        out_type=jax.ShapeDtypeStruct((_NC, n, _MW), jnp.float32),
        scratch_types=[pltpu.VMEM((2, _CH), jnp.int32),
                       pltpu.VMEM((2, _CH, _MW), jnp.float32),
                       pltpu.VMEM((zch, _MW), jnp.float32),
                       pltpu.VMEM_SHARED((n, _MW), jnp.float32),
                       pltpu.SemaphoreType.DMA],
    )
    def k(*refs):
        m_hs = refs[:len(ms)]
        dst_h, p_o, idxd, mbuf, zbuf, acc, semf = refs[len(ms):]
        cid = lax.axis_index("c")
        sid = lax.axis_index("s")
        wid = sid * _NC + cid

        def zrow(r, carry):
            for cc in range(_MW // 16):
                zbuf[r, pl.ds(cc * 16, 16)] = jnp.zeros((16,), jnp.float32)
            return carry

        lax.fori_loop(0, zch, zrow, 0)
        for j in range(nz):
            pltpu.sync_copy(zbuf, acc.at[pl.ds(sid * rows_t + j * zch, zch)])

        @pl.when(sid == 0)
        def _():
            pltpu.sync_copy(zbuf.at[pl.ds(0, extra)],
                            acc.at[pl.ds(_NS * rows_t, extra)])

        plsc.subcore_barrier()

        for q, m_h in enumerate(m_hs):
            ncq = qch[q]
            nte = (ncq // (2 * _NW)) * (2 * _NW)
            ntw = nte // _NW
            nextra = ncq - nte
            goff = qoff[q] * _CH

            def fire(t, b):
                base = (wid + t * _NW) * _CH
                pltpu.async_copy(dst_h.at[pl.ds(goff + base, _CH)],
                                 idxd.at[b], semf)
                pltpu.async_copy(m_h.at[pl.ds(base, _CH)], mbuf.at[b], semf)

            def wait_fire(t, b):
                base = (wid + t * _NW) * _CH
                pltpu.make_async_copy(
                    dst_h.at[pl.ds(goff + base, _CH)], idxd.at[b],
                    semf).wait()
                pltpu.make_async_copy(
                    m_h.at[pl.ds(base, _CH)], mbuf.at[b], semf).wait()

            fire(0, 0)

            def pair(p, carry):
                for b in range(2):
                    t = 2 * p + b

                    @pl.when(t + 1 < ntw)
                    def _():
                        fire(t + 1, 1 - b)

                    wait_fire(t, b)
                    pltpu.sync_copy(mbuf.at[b], acc.at[idxd.at[b]], add=True)
                return carry

            lax.fori_loop(0, ntw // 2, pair, 0)

            @pl.when(wid < nextra)
            def _():
                fire(ntw, 0)
                wait_fire(ntw, 0)
                pltpu.sync_copy(mbuf.at[0], acc.at[idxd.at[0]], add=True)

        plsc.subcore_barrier()
        pltpu.sync_copy(acc.at[pl.ds(sid * rows_t, rows_t)],
                        p_o.at[cid, pl.ds(sid * rows_t, rows_t)])

        @pl.when(sid == 0)
        def _():
            pltpu.sync_copy(acc.at[pl.ds(_NS * rows_t, extra)],
                            p_o.at[cid, pl.ds(_NS * rows_t, extra)])

    return k(*ms, dst)


def _dec_body(x_ref, p_ref, pb_ref, w0x_ref, w0a_ref, w0n_ref, hb0_ref,
              hlng_ref, hlnb_ref, hw1_ref, hb1_ref, o_ref):
    xv = x_ref[...]
    hh = w0a_ref.shape[0]
    p = (p_ref[0] + p_ref[1]) + (pb_ref[0] + pb_ref[1])
    cnt = jnp.maximum(p[:, hh + 2:hh + 3], 1.0)
    agg_h = p[:, 0:hh] / cnt
    agg_v = p[:, hh:hh + 2] / cnt + 1e-8
    mvn = jnp.sqrt(jnp.sum(agg_v * agg_v, axis=1, keepdims=True))
    t = (jnp.dot(xv, w0x_ref[...], preferred_element_type=jnp.float32)
         + jnp.dot(agg_h, w0a_ref[...], preferred_element_type=jnp.float32)
         + mvn * w0n_ref[...] + hb0_ref[...])
    t = jax.nn.softplus(_ln(t, hlng_ref[...], hlnb_ref[...]))
    o_ref[...] = (xv
                  + jnp.dot(t, hw1_ref[...], preferred_element_type=jnp.float32)
                  + hb1_ref[...])


def _decode(x, pa, pb, w0x, w0a, w0n, hb0, hlng, hlnb, hw1, hb1):
    n, d = x.shape
    bn = 2000
    grid = (n // bn,)
    full = lambda a: pl.BlockSpec(a.shape, lambda i: (0,) * a.ndim)
    ws = [w0x, w0a, w0n, hb0, hlng, hlnb, hw1, hb1]
    return pl.pallas_call(
        _dec_body,
        grid=grid,
        in_specs=[pl.BlockSpec((bn, d), lambda i: (i, 0)),
                  pl.BlockSpec((_NC, bn, _MW), lambda i: (0, i, 0)),
                  pl.BlockSpec((_NC, bn, _MW), lambda i: (0, i, 0))] +
                 [full(a) for a in ws],
        out_specs=pl.BlockSpec((bn, d), lambda i: (i, 0)),
        out_shape=jax.ShapeDtypeStruct((n, d), jnp.float32),
    )(x, pa, pb, *ws)


def kernel(x, pos, vel, edge_index,
           e_w0, e_b0, e_ln_g, e_ln_b, e_w1, e_b1, e_w2, e_b2,
           v_w0, v_b0, v_ln_g, v_ln_b, v_w1, v_b1,
           h_w0, h_b0, h_ln_g, h_ln_b, h_w1, h_b1):
    n, d = x.shape
    hh = e_b0.shape[0]
    src = edge_index[0].astype(jnp.int32)
    dst = edge_index[1].astype(jnp.int32)

    wd = jnp.concatenate([e_w0[:d], v_w0[:d]], axis=1)
    ws_ = jnp.concatenate([e_w0[d:2 * d], v_w0[d:2 * d]], axis=1)
    wg = jnp.concatenate([e_w0[2 * d:], v_w0[2 * d:]], axis=1)
    wg = jnp.concatenate([wg, jnp.zeros((3, 2 * hh), jnp.float32)], axis=0)
    bcat = jnp.concatenate([e_b0, v_b0]).reshape(1, 2 * hh)
    pvflat = jnp.concatenate([pos, vel], axis=1).reshape(-1)

    h2 = 2 * hh
    zb = jnp.zeros((hh, hh), jnp.float32)
    jb = jnp.full((hh, hh), 1.0 / hh, jnp.float32)
    mm = jnp.concatenate([jnp.concatenate([jb, zb], 1),
                          jnp.concatenate([zb, jb], 1)], 0)
    lncg = jnp.concatenate([e_ln_g, v_ln_g]).reshape(1, h2)
    lncb = jnp.concatenate([e_ln_b, v_ln_b]).reshape(1, h2)
    vw1p = jnp.concatenate([v_w1, jnp.zeros((hh, hh - 1), jnp.float32)], 1)
    w1c = jnp.concatenate([jnp.concatenate([e_w1, zb], 1),
                           jnp.concatenate([zb, vw1p], 1)], 0)
    b1c = jnp.concatenate(
        [e_b1, v_b1, jnp.zeros((hh - 1,), jnp.float32)]).reshape(1, h2)
    w2c = jnp.concatenate([jnp.concatenate([e_w2, zb], 1),
                           jnp.concatenate([zb, zb], 1)], 0)
    b2c = (jnp.concatenate([e_b2, jnp.zeros((hh,), jnp.float32)])
           .at[hh + 2].set(1.0).reshape(1, h2))
    rpsel = (jnp.zeros((8, h2), jnp.float32)
             .at[5, hh].set(1.0).at[6, hh + 1].set(1.0))

    td, ts = _prep(x, wd, ws_)
    e = src.shape[0]
    ncht = e // _CH
    sl_ch = [640, 640, 640, ncht - 1920]
    nspl = 2
    bounds = [0]
    for c in sl_ch:
        bounds.append(bounds[-1] + c * _CH)
    ms = []
    for q in range(len(sl_ch)):
        lo, hi = bounds[q], bounds[q + 1]
        s_q, geo_q = _sc_gather(td, ts, pvflat, src[lo:hi], dst[lo:hi])
        ms.append(_edge(s_q, geo_q, wg, bcat, mm, lncg, lncb,
                        w1c, b1c, w2c, b2c, rpsel))
    mid = bounds[nspl]
    pa = _sc_scatter(ms[:nspl], dst[:mid], n)
    pb = _sc_scatter(ms[nspl:], dst[mid:], n)
    out = _decode(x, pa, pb, h_w0[:d], h_w0[d:d + hh],
                  h_w0[d + hh:].reshape(1, hh),
                  h_b0.reshape(1, hh), h_ln_g.reshape(1, hh),
                  h_ln_b.reshape(1, hh), h_w1, h_b1.reshape(1, d))
    return out
